# Initial kernel scaffold; baseline (speedup 1.0000x reference)
import jax
import jax.numpy as jnp
from jax.experimental import pallas as pl


def kernel(x, table):
    raise NotImplementedError("write your pallas kernel here")


# speedup vs baseline: 242.7211x; 242.7211x over previous
import functools

import jax
import jax.numpy as jnp
from jax import lax
from jax.experimental import pallas as pl
from jax.experimental.pallas import tpu as pltpu, tpu_sc as plsc

_INFO = plsc.get_sparse_core_info()
_NC = _INFO.num_cores
_NS = _INFO.num_subcores
_NW = _NC * _NS
_L = 16

_TOTAL = 16384 * 200
_PER_W = _TOTAL // _NW
_CHUNK = 12800
_NCHUNK = _PER_W // _CHUNK
_UNROLL = 8


def _lookup_body(x_hbm, table_hbm, out_hbm, table_v, x_v, y_v):
    wid = lax.axis_index("s") * _NC + lax.axis_index("c")
    base = wid * _PER_W

    pltpu.sync_copy(table_hbm, table_v)

    def do_chunk(c, _):
        off = base + c * _CHUNK
        pltpu.sync_copy(x_hbm.at[pl.ds(off, _CHUNK)], x_v)

        def body(i, _):
            pos = i * (_L * _UNROLL)
            for u in range(_UNROLL):
                p = pos + u * _L
                idx = x_v[pl.ds(p, _L)]
                y_v[pl.ds(p, _L)] = plsc.load_gather(table_v, [idx])
            return 0

        lax.fori_loop(0, _CHUNK // (_L * _UNROLL), body, 0)
        pltpu.sync_copy(y_v, out_hbm.at[pl.ds(off, _CHUNK)])
        return 0

    lax.fori_loop(0, _NCHUNK, do_chunk, 0)


@jax.jit
def _lookup(x_flat, table):
    mesh = plsc.VectorSubcoreMesh(core_axis_name="c", subcore_axis_name="s")
    fn = pl.kernel(
        _lookup_body,
        mesh=mesh,
        out_type=jax.ShapeDtypeStruct((_TOTAL,), jnp.float32),
        scratch_types=[
            pltpu.VMEM((256,), jnp.float32),
            pltpu.VMEM((_CHUNK,), jnp.int32),
            pltpu.VMEM((_CHUNK,), jnp.float32),
        ],
        compiler_params=pltpu.CompilerParams(needs_layout_passes=False),
    )
    return fn(x_flat, table)


def kernel(x, table):
    y_flat = _lookup(x.reshape(-1), table)
    return y_flat.reshape(x.shape)


# speedup vs baseline: 303.9240x; 1.2522x over previous
import jax
import jax.numpy as jnp
from jax import lax
from jax.experimental import pallas as pl
from jax.experimental.pallas import tpu as pltpu, tpu_sc as plsc

_INFO = plsc.get_sparse_core_info()
_NC = _INFO.num_cores
_NS = _INFO.num_subcores
_NW = _NC * _NS
_L = 16

_TOTAL = 16384 * 200
_PER_W = _TOTAL // _NW
_CHUNK = 12800
_NCHUNK = _PER_W // _CHUNK


def _lookup_body(x_hbm, table_hbm, out_hbm, table_v,
                 x0, x1, y0, y1, sx0, sx1, sy0, sy1):
    wid = lax.axis_index("s") * _NC + lax.axis_index("c")
    base = wid * _PER_W
    x_bufs, y_bufs = (x0, x1), (y0, y1)
    sx, sy = (sx0, sx1), (sy0, sy1)

    pltpu.sync_copy(table_hbm, table_v)

    def start_x(c):
        b = c % 2
        return pltpu.async_copy(
            x_hbm.at[pl.ds(base + c * _CHUNK, _CHUNK)], x_bufs[b], sx[b])

    xc = [None, None]
    yc = [None, None]
    xc[0] = start_x(0)
    for c in range(_NCHUNK):
        b = c % 2
        if c + 1 < _NCHUNK:
            xc[(c + 1) % 2] = start_x(c + 1)
        xc[b].wait()
        if c >= 2:
            yc[b].wait()
        x_v, y_v = x_bufs[b], y_bufs[b]

        @plsc.parallel_loop(0, _CHUNK, step=_L, unroll=8)
        def _gather(i):
            y_v[pl.ds(i, _L)] = plsc.load_gather(table_v, [x_v[pl.ds(i, _L)]])

        yc[b] = pltpu.async_copy(
            y_v, out_hbm.at[pl.ds(base + c * _CHUNK, _CHUNK)], sy[b])

    yc[(_NCHUNK - 2) % 2].wait()
    yc[(_NCHUNK - 1) % 2].wait()


@jax.jit
def _lookup(x_flat, table):
    mesh = plsc.VectorSubcoreMesh(core_axis_name="c", subcore_axis_name="s")
    fn = pl.kernel(
        _lookup_body,
        mesh=mesh,
        out_type=jax.ShapeDtypeStruct((_TOTAL,), jnp.float32),
        scratch_types=[
            pltpu.VMEM((256,), jnp.float32),
            pltpu.VMEM((_CHUNK,), jnp.int32),
            pltpu.VMEM((_CHUNK,), jnp.int32),
            pltpu.VMEM((_CHUNK,), jnp.float32),
            pltpu.VMEM((_CHUNK,), jnp.float32),
            pltpu.SemaphoreType.DMA,
            pltpu.SemaphoreType.DMA,
            pltpu.SemaphoreType.DMA,
            pltpu.SemaphoreType.DMA,
        ],
        compiler_params=pltpu.CompilerParams(needs_layout_passes=False),
    )
    return fn(x_flat, table)


def kernel(x, table):
    y_flat = _lookup(x.reshape(-1), table)
    return y_flat.reshape(x.shape)


# speedup vs baseline: 525.8804x; 1.7303x over previous
import jax
import jax.numpy as jnp
from jax import lax
from jax.experimental import pallas as pl
from jax.experimental.pallas import tpu as pltpu, tpu_sc as plsc

_INFO = plsc.get_sparse_core_info()
_NC = _INFO.num_cores
_NS = _INFO.num_subcores
_NW = _NC * _NS
_L = 16

_ROWS = 16384
_COLS = 200
_RPW = _ROWS // _NW
_RCHUNK = 64
_NCHUNK = _RPW // _RCHUNK
_COFFS = tuple(range(0, _COLS - _L, _L)) + (_COLS - _L,)


def _lookup_body(x_hbm, table_hbm, out_hbm, table_v,
                 x0, x1, y0, y1, sx0, sx1, sy0, sy1):
    wid = lax.axis_index("s") * _NC + lax.axis_index("c")
    base = wid * _RPW
    x_bufs, y_bufs = (x0, x1), (y0, y1)
    sx, sy = (sx0, sx1), (sy0, sy1)

    pltpu.sync_copy(table_hbm, table_v)

    def start_x(c):
        b = c % 2
        return pltpu.async_copy(
            x_hbm.at[pl.ds(base + c * _RCHUNK, _RCHUNK)], x_bufs[b], sx[b])

    xc = [None, None]
    yc = [None, None]
    xc[0] = start_x(0)
    for c in range(_NCHUNK):
        b = c % 2
        if c + 1 < _NCHUNK:
            xc[(c + 1) % 2] = start_x(c + 1)
        xc[b].wait()
        if c >= 2:
            yc[b].wait()
        x_v, y_v = x_bufs[b], y_bufs[b]

        @plsc.parallel_loop(0, _RCHUNK, step=1, unroll=2)
        def _gather(r):
            for c0 in _COFFS:
                idx = x_v[r, pl.ds(c0, _L)]
                y_v[r, pl.ds(c0, _L)] = plsc.load_gather(table_v, [idx])

        yc[b] = pltpu.async_copy(
            y_v, out_hbm.at[pl.ds(base + c * _RCHUNK, _RCHUNK)], sy[b])

    yc[(_NCHUNK - 2) % 2].wait()
    yc[(_NCHUNK - 1) % 2].wait()


@jax.jit
def kernel(x, table):
    mesh = plsc.VectorSubcoreMesh(core_axis_name="c", subcore_axis_name="s")
    fn = pl.kernel(
        _lookup_body,
        mesh=mesh,
        out_type=jax.ShapeDtypeStruct((_ROWS, _COLS), jnp.float32),
        scratch_types=[
            pltpu.VMEM((256,), jnp.float32),
            pltpu.VMEM((_RCHUNK, _COLS), jnp.int32),
            pltpu.VMEM((_RCHUNK, _COLS), jnp.int32),
            pltpu.VMEM((_RCHUNK, _COLS), jnp.float32),
            pltpu.VMEM((_RCHUNK, _COLS), jnp.float32),
            pltpu.SemaphoreType.DMA,
            pltpu.SemaphoreType.DMA,
            pltpu.SemaphoreType.DMA,
            pltpu.SemaphoreType.DMA,
        ],
        compiler_params=pltpu.CompilerParams(needs_layout_passes=False),
    )
    return fn(x, table)


# speedup vs baseline: 909.2968x; 1.7291x over previous
import jax
import jax.numpy as jnp
from jax import lax
from jax.experimental import pallas as pl
from jax.experimental.pallas import tpu as pltpu, tpu_sc as plsc

_INFO = plsc.get_sparse_core_info()
_NC = _INFO.num_cores
_NS = _INFO.num_subcores
_NW = _NC * _NS
_L = 16

_F = 200
_B = 16384
_CPW = _B // _NW
_RCHUNK = 40
_NCHUNK = _F // _RCHUNK


def _lookup_body(x_hbm, table_hbm, out_hbm, table_v,
                 x0, x1, y0, y1, sx0, sx1, sy0, sy1):
    wid = lax.axis_index("s") * _NC + lax.axis_index("c")
    col0 = wid * _CPW
    x_bufs, y_bufs = (x0, x1), (y0, y1)
    sx, sy = (sx0, sx1), (sy0, sy1)

    pltpu.sync_copy(table_hbm, table_v)

    def start_x(c):
        b = c % 2
        return pltpu.async_copy(
            x_hbm.at[pl.ds(c * _RCHUNK, _RCHUNK), pl.ds(col0, _CPW)],
            x_bufs[b], sx[b])

    xc = [None, None]
    yc = [None, None]
    xc[0] = start_x(0)
    for c in range(_NCHUNK):
        b = c % 2
        if c + 1 < _NCHUNK:
            xc[(c + 1) % 2] = start_x(c + 1)
        xc[b].wait()
        if c >= 2:
            yc[b].wait()
        x_v, y_v = x_bufs[b], y_bufs[b]

        @plsc.parallel_loop(0, _RCHUNK, step=1)
        def _gather(r):
            for cs in range(0, _CPW, _L):
                idx = x_v[r, pl.ds(cs, _L)]
                y_v[r, pl.ds(cs, _L)] = plsc.load_gather(table_v, [idx])

        yc[b] = pltpu.async_copy(
            y_v,
            out_hbm.at[pl.ds(c * _RCHUNK, _RCHUNK), pl.ds(col0, _CPW)],
            sy[b])

    yc[(_NCHUNK - 2) % 2].wait()
    yc[(_NCHUNK - 1) % 2].wait()


@jax.jit
def kernel(x, table):
    mesh = plsc.VectorSubcoreMesh(core_axis_name="c", subcore_axis_name="s")
    fn = pl.kernel(
        _lookup_body,
        mesh=mesh,
        out_type=jax.ShapeDtypeStruct((_F, _B), jnp.float32),
        scratch_types=[
            pltpu.VMEM((256,), jnp.float32),
            pltpu.VMEM((_RCHUNK, _CPW), jnp.int32),
            pltpu.VMEM((_RCHUNK, _CPW), jnp.int32),
            pltpu.VMEM((_RCHUNK, _CPW), jnp.float32),
            pltpu.VMEM((_RCHUNK, _CPW), jnp.float32),
            pltpu.SemaphoreType.DMA,
            pltpu.SemaphoreType.DMA,
            pltpu.SemaphoreType.DMA,
            pltpu.SemaphoreType.DMA,
        ],
        compiler_params=pltpu.CompilerParams(needs_layout_passes=False),
    )
    return fn(x.T, table).T


# speedup vs baseline: 950.9630x; 1.0458x over previous
import jax
import jax.numpy as jnp
from jax import lax
from jax.experimental import pallas as pl
from jax.experimental.pallas import tpu as pltpu, tpu_sc as plsc

_INFO = plsc.get_sparse_core_info()
_NC = _INFO.num_cores
_NS = _INFO.num_subcores
_NW = _NC * _NS
_L = 16

_F = 200
_B = 16384
_CPW = _B // _NW
_CHUNKS = (8, 40, 48, 48, 40, 16)
_RMAX = max(_CHUNKS)
_STARTS = tuple(sum(_CHUNKS[:i]) for i in range(len(_CHUNKS)))


def _lookup_body(x_hbm, table_hbm, out_hbm, table_v,
                 x0, x1, y0, y1, st, sx0, sx1, sy0, sy1):
    wid = lax.axis_index("s") * _NC + lax.axis_index("c")
    col0 = wid * _CPW
    x_bufs, y_bufs = (x0, x1), (y0, y1)
    sx, sy = (sx0, sx1), (sy0, sy1)

    def start_x(c):
        b = c % 2
        return pltpu.async_copy(
            x_hbm.at[pl.ds(_STARTS[c], _CHUNKS[c]), pl.ds(col0, _CPW)],
            x_bufs[b].at[pl.ds(0, _CHUNKS[c])], sx[b])

    xc = [None, None]
    yc = [None, None]
    xc[0] = start_x(0)
    tc = pltpu.async_copy(table_hbm, table_v, st)
    xc[1] = start_x(1)
    tc.wait()
    for c in range(len(_CHUNKS)):
        b = c % 2
        xc[b].wait()
        if c >= 2:
            yc[b].wait()
        x_v, y_v = x_bufs[b], y_bufs[b]

        @plsc.parallel_loop(0, _CHUNKS[c], step=1)
        def _gather(r):
            for cs in range(0, _CPW, _L):
                idx = x_v[r, pl.ds(cs, _L)]
                y_v[r, pl.ds(cs, _L)] = plsc.load_gather(table_v, [idx])

        yc[b] = pltpu.async_copy(
            y_v.at[pl.ds(0, _CHUNKS[c])],
            out_hbm.at[pl.ds(_STARTS[c], _CHUNKS[c]), pl.ds(col0, _CPW)],
            sy[b])
        if c + 2 < len(_CHUNKS):
            xc[b] = start_x(c + 2)

    yc[len(_CHUNKS) % 2].wait()
    yc[(len(_CHUNKS) - 1) % 2].wait()


@jax.jit
def kernel(x, table):
    mesh = plsc.VectorSubcoreMesh(core_axis_name="c", subcore_axis_name="s")
    fn = pl.kernel(
        _lookup_body,
        mesh=mesh,
        out_type=jax.ShapeDtypeStruct((_F, _B), jnp.float32),
        scratch_types=[
            pltpu.VMEM((256,), jnp.float32),
            pltpu.VMEM((_RMAX, _CPW), jnp.int32),
            pltpu.VMEM((_RMAX, _CPW), jnp.int32),
            pltpu.VMEM((_RMAX, _CPW), jnp.float32),
            pltpu.VMEM((_RMAX, _CPW), jnp.float32),
            pltpu.SemaphoreType.DMA,
            pltpu.SemaphoreType.DMA,
            pltpu.SemaphoreType.DMA,
            pltpu.SemaphoreType.DMA,
            pltpu.SemaphoreType.DMA,
        ],
        compiler_params=pltpu.CompilerParams(needs_layout_passes=False),
    )
    return fn(x.T, table).T
